# baseline (speedup 1.0000x reference)
import functools

import jax
import jax.numpy as jnp
from jax import lax
from jax.experimental import pallas as pl
from jax.experimental.pallas import tpu as pltpu
from jax.experimental.pallas import tpu_sc as plsc

_L = 16


def _sae_sc(words, energies_adj, table32, B, W, num_workers):
    rows_per_w = B // num_workers
    C = 128
    n_chunks = rows_per_w // C
    mesh = plsc.VectorSubcoreMesh(core_axis_name="c", subcore_axis_name="s")
    NC = mesh.num_cores

    @functools.partial(
        pl.kernel,
        out_type=jax.ShapeDtypeStruct((B,), jnp.float32),
        mesh=mesh,
        compiler_params=pltpu.CompilerParams(needs_layout_passes=False),
        scratch_types=[
            pltpu.VMEM((64,), jnp.float32),
            pltpu.VMEM((C * W,), jnp.int32),
            pltpu.VMEM((C * W,), jnp.int32),
            pltpu.VMEM((rows_per_w,), jnp.float32),
            pltpu.VMEM((rows_per_w,), jnp.float32),
            pltpu.SemaphoreType.DMA,
            pltpu.SemaphoreType.DMA,
            pltpu.SemaphoreType.DMA,
        ],
    )
    def k(words_hbm, en_hbm, table_hbm, out_hbm,
          table_v, data0, data1, en_v, out_v, sem0, sem1, semt):
        wid = lax.axis_index("s") * NC + lax.axis_index("c")
        base_row = wid * rows_per_w
        base_word = base_row * W

        pltpu.sync_copy(table_hbm, table_v)
        en_cp = pltpu.async_copy(en_hbm.at[pl.ds(base_row, rows_per_w)],
                                 en_v, semt)

        bufs = (data0, data1)
        sems = (sem0, sem1)
        copies = [None] * n_chunks
        copies[0] = pltpu.async_copy(
            words_hbm.at[pl.ds(base_word, C * W)], bufs[0], sems[0])

        for g in range(n_chunks):
            if g + 1 < n_chunks:
                copies[g + 1] = pltpu.async_copy(
                    words_hbm.at[pl.ds(base_word + (g + 1) * C * W, C * W)],
                    bufs[(g + 1) % 2], sems[(g + 1) % 2])
            copies[g].wait()
            buf = bufs[g % 2]

            for rb in range(C // _L):
                base_vec = (lax.broadcasted_iota(jnp.int32, (_L,), 0)
                            + rb * _L) * W

                def w_body(w, carry, buf=buf):
                    acc, addr = carry
                    idx16 = plsc.load_gather(buf, [addr])
                    se = plsc.load_gather(table_v, [idx16])
                    return acc + se, addr + jnp.int32(1)

                acc, _ = lax.fori_loop(
                    0, W, w_body,
                    (jnp.zeros((_L,), jnp.float32), base_vec))
                out_v[pl.ds(g * C + rb * _L, _L)] = acc

        en_cp.wait()
        for v in range(rows_per_w // _L):
            sl = pl.ds(v * _L, _L)
            out_v[sl] = out_v[sl] + en_v[sl]
        pltpu.sync_copy(out_v, out_hbm.at[pl.ds(base_row, rows_per_w)])

    return k(words, energies_adj, table32)


def kernel(species, energies, self_energies):
    B, A = species.shape
    W = 2 * A
    num_workers = 32

    words = lax.bitcast_convert_type(species, jnp.int32).reshape(B * W)
    table32 = self_energies.astype(jnp.float32)
    energies_adj = energies.astype(jnp.float32) - jnp.float32(A) * table32[0]

    sae32 = _sae_sc(words, energies_adj, table32, B, W, num_workers)

    out_dtype = jnp.result_type(energies.dtype, self_energies.dtype)
    return (species, sae32.astype(out_dtype))


# speedup vs baseline: 9.8657x; 9.8657x over previous
import functools

import jax
import jax.numpy as jnp
from jax import lax
from jax.experimental import pallas as pl
from jax.experimental.pallas import tpu as pltpu
from jax.experimental.pallas import tpu_sc as plsc

_L = 16
_M = 128


def _sae_sc(words2d, en2d, table2d, B, W, num_workers):
    rows_per_w = B // num_workers
    C = 128
    n_chunks = rows_per_w // C
    hrows_chunk = C * W // _M
    hrows_w = rows_per_w * W // _M
    erows_w = rows_per_w // _M
    mesh = plsc.VectorSubcoreMesh(core_axis_name="c", subcore_axis_name="s")
    NC = mesh.num_cores

    @functools.partial(
        pl.kernel,
        out_type=jax.ShapeDtypeStruct((B // _M, _M), jnp.float32),
        mesh=mesh,
        compiler_params=pltpu.CompilerParams(needs_layout_passes=False),
        scratch_types=[
            pltpu.VMEM((8, _M), jnp.float32),
            pltpu.VMEM((hrows_chunk, _M), jnp.int32),
            pltpu.VMEM((hrows_chunk, _M), jnp.int32),
            pltpu.VMEM((erows_w, _M), jnp.float32),
            pltpu.VMEM((erows_w, _M), jnp.float32),
            pltpu.SemaphoreType.DMA,
            pltpu.SemaphoreType.DMA,
            pltpu.SemaphoreType.DMA,
        ],
    )
    def k(words_hbm, en_hbm, table_hbm, out_hbm,
          table_v, data0, data1, en_v, out_v, sem0, sem1, semt):
        wid = lax.axis_index("s") * NC + lax.axis_index("c")
        hbase = wid * hrows_w

        pltpu.sync_copy(table_hbm, table_v)
        en_cp = pltpu.async_copy(en_hbm.at[pl.ds(wid * erows_w, erows_w)],
                                 en_v, semt)

        zero_row = jnp.zeros((_L,), jnp.int32)
        bufs = (data0, data1)
        sems = (sem0, sem1)
        copies = [None] * n_chunks
        copies[0] = pltpu.async_copy(
            words_hbm.at[pl.ds(hbase, hrows_chunk)], bufs[0], sems[0])

        for g in range(n_chunks):
            if g + 1 < n_chunks:
                copies[g + 1] = pltpu.async_copy(
                    words_hbm.at[pl.ds(hbase + (g + 1) * hrows_chunk,
                                       hrows_chunk)],
                    bufs[(g + 1) % 2], sems[(g + 1) % 2])
            copies[g].wait()
            buf = bufs[g % 2]

            for rb in range(C // _L):
                base_vec = (lax.broadcasted_iota(jnp.int32, (_L,), 0)
                            + rb * _L) * W

                def w_body(w, carry, buf=buf):
                    acc, addr = carry
                    idx16 = plsc.load_gather(
                        buf, [lax.shift_right_logical(addr, jnp.int32(7)),
                              lax.bitwise_and(addr, jnp.int32(_M - 1))])
                    se = plsc.load_gather(table_v, [zero_row, idx16])
                    return acc + se, addr + jnp.int32(1)

                acc, _ = lax.fori_loop(
                    0, W, w_body,
                    (jnp.zeros((_L,), jnp.float32), base_vec),
                    unroll=16)
                off = g * C + rb * _L
                out_v[off // _M, pl.ds(off % _M, _L)] = acc

        en_cp.wait()
        for v in range(rows_per_w // _L):
            r, c = (v * _L) // _M, (v * _L) % _M
            out_v[r, pl.ds(c, _L)] = (out_v[r, pl.ds(c, _L)]
                                      + en_v[r, pl.ds(c, _L)])
        pltpu.sync_copy(out_v, out_hbm.at[pl.ds(wid * erows_w, erows_w)])

    return k(words2d, en2d, table2d)


def kernel(species, energies, self_energies):
    B, A = species.shape
    W = A
    num_workers = 32

    words2d = species.astype(jnp.int32).reshape(B * W // _M, _M)
    en2d = energies.astype(jnp.float32).reshape(B // _M, _M)
    table32 = self_energies.astype(jnp.float32)
    table2d = jnp.concatenate(
        [table32, jnp.zeros((8 * _M - table32.shape[0],), jnp.float32)]
    ).reshape(8, _M)

    sae2d = _sae_sc(words2d, en2d, table2d, B, W, num_workers)

    out_dtype = jnp.result_type(energies.dtype, self_energies.dtype)
    return (species, sae2d.reshape(B).astype(out_dtype))
